# baseline (speedup 1.0000x reference)
import functools

import jax
import jax.numpy as jnp
from jax.experimental import pallas as pl
from jax.experimental.pallas import tpu as pltpu
from jax.experimental.pallas import tpu_sc as plsc

_B = 1
_S = 2048
_DM = 768
_DFF = 3072
_E = 8
_RANK = 16
_TOPK = 2
_TS = 512
_SC_BLK = 128
_SC_LANES = 16


def _gate_body(h_ref, gw_ref, gb_ref, lt_ref):
    hb = h_ref[...].astype(jnp.bfloat16)
    gwb = gw_ref[...].astype(jnp.bfloat16)
    lt_ref[...] = jax.lax.dot_general(
        gwb, hb, (((1,), (1,)), ((), ())),
        preferred_element_type=jnp.float32) + gb_ref[...]


def _gate(h2d, gate_w, gate_b_col):
    return pl.pallas_call(
        _gate_body,
        grid=(_S // _TS,),
        in_specs=[
            pl.BlockSpec((_TS, _DM), lambda i: (i, 0)),
            pl.BlockSpec((_E, _DM), lambda i: (0, 0)),
            pl.BlockSpec((_E, 1), lambda i: (0, 0)),
        ],
        out_specs=pl.BlockSpec((_E, _TS), lambda i: (0, i)),
        out_shape=jax.ShapeDtypeStruct((_E, _S), jnp.float32),
        compiler_params=pltpu.CompilerParams(
            dimension_semantics=("arbitrary",),
        ),
    )(h2d, gate_w, gate_b_col)


def _route_sc_block(lt_vmem, s1_vmem, s2_vmem):
    neg_big = jnp.float32(-3e38)
    for c in range(0, _SC_BLK, _SC_LANES):
        xs = [lt_vmem[e, pl.ds(c, _SC_LANES)] for e in range(_E)]
        m1 = functools.reduce(jnp.maximum, xs)
        notf = jnp.ones((_SC_LANES,), jnp.float32)
        sel1 = []
        for e in range(_E):
            is1 = jnp.where(xs[e] == m1, notf, jnp.float32(0.0))
            notf = notf - is1
            sel1.append(is1)
            s1_vmem[e, pl.ds(c, _SC_LANES)] = is1
        x2 = [jnp.where(sel1[e] > 0, neg_big, xs[e]) for e in range(_E)]
        m2 = functools.reduce(jnp.maximum, x2)
        notf2 = jnp.ones((_SC_LANES,), jnp.float32)
        for e in range(_E):
            is2 = jnp.where(x2[e] == m2, notf2, jnp.float32(0.0))
            notf2 = notf2 - is2
            s2_vmem[e, pl.ds(c, _SC_LANES)] = is2


def _route(lt):
    out_t = jax.ShapeDtypeStruct((_E, _S), jnp.float32)
    mesh = plsc.VectorSubcoreMesh(core_axis_name="core",
                                  subcore_axis_name="subcore")

    @pl.kernel(out_type=[out_t, out_t], mesh=mesh, scratch_types=[])
    def route_kernel(lt_hbm, s1_hbm, s2_hbm):
        pltpu.emit_pipeline(
            _route_sc_block,
            grid=(_S // _SC_BLK,),
            in_specs=[pl.BlockSpec((_E, _SC_BLK), index_map=lambda i: (0, i))],
            out_specs=[pl.BlockSpec((_E, _SC_BLK), index_map=lambda i: (0, i)),
                       pl.BlockSpec((_E, _SC_BLK), index_map=lambda i: (0, i))],
            core_axis_name=("core", "subcore"),
            dimension_semantics=(pltpu.PARALLEL,),
        )(lt_hbm, s1_hbm, s2_hbm)

    return route_kernel(lt)


def _proj_body(h_ref, wi_ref, a_ref, sh_ref, p_ref, wib_ref, ab_ref):
    @pl.when(pl.program_id(0) == 0)
    def _pack():
        wib_ref[...] = wi_ref[...].astype(jnp.bfloat16)
        ab_ref[...] = a_ref[...].astype(jnp.bfloat16)

    hb = h_ref[...].astype(jnp.bfloat16)
    sh_ref[...] = jax.lax.dot_general(
        hb, wib_ref[...], (((1,), (1,)), ((), ())),
        preferred_element_type=jnp.float32).astype(jnp.bfloat16)
    p_ref[...] = jax.lax.dot_general(
        hb, ab_ref[...], (((1,), (1,)), ((), ())),
        preferred_element_type=jnp.float32)


def _proj(h2d, wi_w, a_all):
    return pl.pallas_call(
        _proj_body,
        grid=(_S // _TS,),
        in_specs=[
            pl.BlockSpec((_TS, _DM), lambda i: (i, 0)),
            pl.BlockSpec((_DFF, _DM), lambda i: (0, 0)),
            pl.BlockSpec((_E * _RANK, _DM), lambda i: (0, 0)),
        ],
        out_specs=[
            pl.BlockSpec((_TS, _DFF), lambda i: (i, 0)),
            pl.BlockSpec((_TS, _E * _RANK), lambda i: (i, 0)),
        ],
        out_shape=[
            jax.ShapeDtypeStruct((_S, _DFF), jnp.bfloat16),
            jax.ShapeDtypeStruct((_S, _E * _RANK), jnp.float32),
        ],
        scratch_shapes=[
            pltpu.VMEM((_DFF, _DM), jnp.bfloat16),
            pltpu.VMEM((_E * _RANK, _DM), jnp.bfloat16),
        ],
        compiler_params=pltpu.CompilerParams(
            dimension_semantics=("arbitrary",),
        ),
    )(h2d, wi_w, a_all)


def _comb_body(sh_ref, p_ref, lt_ref, s1_ref, s2_ref, wo_ref, ball_ref,
               out_ref, wob_ref, ballb_ref):
    @pl.when(pl.program_id(0) == 0)
    def _pack():
        wob_ref[...] = wo_ref[...].astype(jnp.bfloat16)
        ballb_ref[...] = ball_ref[...].astype(jnp.bfloat16)

    logits = jnp.transpose(lt_ref[...])
    sel1 = jnp.transpose(s1_ref[...])
    sel2 = jnp.transpose(s2_ref[...])
    m = jnp.max(logits, axis=-1, keepdims=True)
    ex = jnp.exp(logits - m)
    s = ex / jnp.sum(ex, axis=-1, keepdims=True)
    v1 = jnp.sum(s * sel1, axis=-1, keepdims=True)
    v2 = jnp.sum(s * sel2, axis=-1, keepdims=True)
    colf = jax.lax.broadcasted_iota(jnp.int32, (_TS, _E), 1).astype(jnp.float32)
    i1 = jnp.sum(colf * sel1, axis=-1, keepdims=True).astype(jnp.int32)
    i2 = jnp.sum(colf * sel2, axis=-1, keepdims=True).astype(jnp.int32)

    shared = sh_ref[...].astype(jnp.float32)
    p = p_ref[...]

    pexp = jax.lax.broadcasted_iota(jnp.int32, p.shape, 1) // _RANK
    q1 = jnp.where(pexp == i1, p, 0.0).astype(jnp.bfloat16)
    q2 = jnp.where(pexp == i2, p, 0.0).astype(jnp.bfloat16)
    l1 = jax.lax.dot_general(
        q1, ballb_ref[...], (((1,), (0,)), ((), ())),
        preferred_element_type=jnp.float32)
    l2 = jax.lax.dot_general(
        q2, ballb_ref[...], (((1,), (0,)), ((), ())),
        preferred_element_type=jnp.float32)

    acc = v1 * jnp.maximum(shared + l1, 0.0) + v2 * jnp.maximum(shared + l2, 0.0)

    out_ref[...] = jax.lax.dot_general(
        acc.astype(jnp.bfloat16), wob_ref[...], (((1,), (1,)), ((), ())),
        preferred_element_type=jnp.float32)


def _comb(sh, p, lt, s1, s2, wo_w, ball):
    return pl.pallas_call(
        _comb_body,
        grid=(_S // _TS,),
        in_specs=[
            pl.BlockSpec((_TS, _DFF), lambda i: (i, 0)),
            pl.BlockSpec((_TS, _E * _RANK), lambda i: (i, 0)),
            pl.BlockSpec((_E, _TS), lambda i: (0, i)),
            pl.BlockSpec((_E, _TS), lambda i: (0, i)),
            pl.BlockSpec((_E, _TS), lambda i: (0, i)),
            pl.BlockSpec((_DM, _DFF), lambda i: (0, 0)),
            pl.BlockSpec((_E * _RANK, _DFF), lambda i: (0, 0)),
        ],
        out_specs=pl.BlockSpec((_TS, _DM), lambda i: (i, 0)),
        out_shape=jax.ShapeDtypeStruct((_S, _DM), jnp.float32),
        scratch_shapes=[
            pltpu.VMEM((_DM, _DFF), jnp.bfloat16),
            pltpu.VMEM((_E * _RANK, _DFF), jnp.bfloat16),
        ],
        compiler_params=pltpu.CompilerParams(
            dimension_semantics=("arbitrary",),
        ),
    )(sh, p, lt, s1, s2, wo_w, ball)


@jax.jit
def _pipeline(h2d, gate_w, gate_b_col, wi_w, wo_w, a_all, ball):
    lt = _gate(h2d, gate_w, gate_b_col)
    s1, s2 = _route(lt)
    sh, p = _proj(h2d, wi_w, a_all)
    return _comb(sh, p, lt, s1, s2, wo_w, ball)


def kernel(hidden_states, gate_w, gate_b, wi_w, wo_w, lora_A, lora_B):
    h2d = hidden_states.reshape(_S, _DM)
    gate_b_col = gate_b.reshape(_E, 1)
    a_all = lora_A.reshape(_E * _RANK, _DM)
    ball = jnp.transpose(lora_B, (0, 2, 1)).reshape(_E * _RANK, _DFF)
    out = _pipeline(h2d, gate_w, gate_b_col, wi_w, wo_w, a_all, ball)
    return out.reshape(_B, _S, _DM)


# speedup vs baseline: 1.5798x; 1.5798x over previous
import functools

import jax
import jax.numpy as jnp
from jax.experimental import pallas as pl
from jax.experimental.pallas import tpu as pltpu

_B = 1
_S = 2048
_DM = 768
_DFF = 3072
_E = 8
_RANK = 16
_TOPK = 2
_TS = 512


def _moe_body(h_ref, gw_ref, gb_ref, wi_ref, wo_hbm, a_ref, ball_ref, out_ref,
              gwb_ref, wib_ref, wob_ref, ab_ref, ballb_ref, wof_ref, wo_sem):
    @pl.when(pl.program_id(0) == 0)
    def _pack():
        pltpu.make_async_copy(wo_hbm, wof_ref, wo_sem).start()
        gwb_ref[...] = gw_ref[...].astype(jnp.bfloat16)
        wib_ref[...] = wi_ref[...].astype(jnp.bfloat16)
        ab_ref[...] = a_ref[...].astype(jnp.bfloat16)
        ballb_ref[...] = ball_ref[...].astype(jnp.bfloat16)

    h = h_ref[...]
    hb = h.astype(jnp.bfloat16)

    logits = jax.lax.dot_general(
        hb, gwb_ref[...], (((1,), (1,)), ((), ())),
        preferred_element_type=jnp.float32) + gb_ref[...]
    m = jnp.max(logits, axis=-1, keepdims=True)
    ex = jnp.exp(logits - m)
    s = ex / jnp.sum(ex, axis=-1, keepdims=True)
    col = jax.lax.broadcasted_iota(jnp.int32, s.shape, 1)
    v1 = jnp.max(s, axis=-1, keepdims=True)
    i1 = jnp.min(jnp.where(s == v1, col, _E), axis=-1, keepdims=True)
    s2 = jnp.where(col == i1, -jnp.inf, s)
    v2 = jnp.max(s2, axis=-1, keepdims=True)
    i2 = jnp.min(jnp.where(s2 == v2, col, _E), axis=-1, keepdims=True)

    shared = jax.lax.dot_general(
        hb, wib_ref[...], (((1,), (1,)), ((), ())),
        preferred_element_type=jnp.float32)
    p = jax.lax.dot_general(
        hb, ab_ref[...], (((1,), (1,)), ((), ())),
        preferred_element_type=jnp.float32)

    pexp = jax.lax.broadcasted_iota(jnp.int32, p.shape, 1) // _RANK
    q1 = jnp.where(pexp == i1, p, 0.0).astype(jnp.bfloat16)
    q2 = jnp.where(pexp == i2, p, 0.0).astype(jnp.bfloat16)
    l1 = jax.lax.dot_general(
        q1, ballb_ref[...], (((1,), (0,)), ((), ())),
        preferred_element_type=jnp.float32)
    l2 = jax.lax.dot_general(
        q2, ballb_ref[...], (((1,), (0,)), ((), ())),
        preferred_element_type=jnp.float32)

    acc = v1 * jnp.maximum(shared + l1, 0.0) + v2 * jnp.maximum(shared + l2, 0.0)

    @pl.when(pl.program_id(0) == 0)
    def _finish_wo():
        pltpu.make_async_copy(wo_hbm, wof_ref, wo_sem).wait()
        wob_ref[...] = wof_ref[...].astype(jnp.bfloat16)

    out_ref[...] = jax.lax.dot_general(
        acc.astype(jnp.bfloat16), wob_ref[...], (((1,), (1,)), ((), ())),
        preferred_element_type=jnp.float32)


@functools.partial(jax.jit, static_argnames=())
def _moe(h2d, gate_w, gate_b2d, wi_w, wo_w, a_all, ball):
    grid = (_S // _TS,)
    return pl.pallas_call(
        _moe_body,
        grid=grid,
        in_specs=[
            pl.BlockSpec((_TS, _DM), lambda i: (i, 0)),
            pl.BlockSpec((_E, _DM), lambda i: (0, 0)),
            pl.BlockSpec((1, _E), lambda i: (0, 0)),
            pl.BlockSpec((_DFF, _DM), lambda i: (0, 0)),
            pl.BlockSpec(memory_space=pl.ANY),
            pl.BlockSpec((_E * _RANK, _DM), lambda i: (0, 0)),
            pl.BlockSpec((_E * _RANK, _DFF), lambda i: (0, 0)),
        ],
        out_specs=pl.BlockSpec((_TS, _DM), lambda i: (i, 0)),
        out_shape=jax.ShapeDtypeStruct((_S, _DM), jnp.float32),
        scratch_shapes=[
            pltpu.VMEM((_E, _DM), jnp.bfloat16),
            pltpu.VMEM((_DFF, _DM), jnp.bfloat16),
            pltpu.VMEM((_DM, _DFF), jnp.bfloat16),
            pltpu.VMEM((_E * _RANK, _DM), jnp.bfloat16),
            pltpu.VMEM((_E * _RANK, _DFF), jnp.bfloat16),
            pltpu.VMEM((_DM, _DFF), jnp.float32),
            pltpu.SemaphoreType.DMA,
        ],
        compiler_params=pltpu.CompilerParams(
            dimension_semantics=("arbitrary",),
        ),
    )(h2d, gate_w, gate_b2d, wi_w, wo_w, a_all, ball)


def kernel(hidden_states, gate_w, gate_b, wi_w, wo_w, lora_A, lora_B):
    h2d = hidden_states.reshape(_S, _DM)
    gate_b2d = gate_b.reshape(1, _E)
    a_all = lora_A.reshape(_E * _RANK, _DM)
    ball = jnp.transpose(lora_B, (0, 2, 1)).reshape(_E * _RANK, _DFF)
    out = _moe(h2d, gate_w, gate_b2d, wi_w, wo_w, a_all, ball)
    return out.reshape(_B, _S, _DM)
